# baseline (speedup 1.0000x reference)
import functools

import jax
import jax.numpy as jnp
from jax import lax
from jax.experimental import pallas as pl
from jax.experimental.pallas import tpu as pltpu
from jax.experimental.pallas import tpu_sc as plsc

N_FRAMES = 4096
K_TAB = 16
D_ROW = 18 * 512
REPLICAS = 8


import numpy as np

_I32_MIN = np.int32(-2147483648)
_RANKS = (1023, 1024, 3071, 3072)


def _key_to_float(v):
  b = jnp.where(v >= 0, v, _I32_MIN - v)
  return lax.bitcast_convert_type(b, jnp.float32)


def _index_kernel(pitch_ref, idx_ref):
  x = pitch_ref[...]
  b = lax.bitcast_convert_type(x, jnp.int32)
  k = jnp.where(b >= 0, b, _I32_MIN - b)

  stats = []
  for r in _RANKS:
    r = jnp.int32(r)
    cnt0 = jnp.sum((k < 0).astype(jnp.int32))
    ans = jnp.where(cnt0 <= r, jnp.int32(0), _I32_MIN)
    for bit in range(30, -1, -1):
      t = ans + jnp.int32(1 << bit)
      cnt = jnp.sum((k < t).astype(jnp.int32))
      ans = jnp.where(cnt <= r, t, ans)
    stats.append(_key_to_float(ans))
  s1023, s1024, s3071, s3072 = stats

  low = s1023 * jnp.float32(0.25) + s1024 * jnp.float32(0.75)
  high = s3071 * jnp.float32(0.75) + s3072 * jnp.float32(0.25)

  pt = x - low
  pt = pt / high
  pt = pt * jnp.float32(16.0)
  m = pt - jnp.float32(16.0) * jnp.floor(pt * jnp.float32(0.0625))
  idx = jnp.round(m).astype(jnp.int32) % K_TAB
  rep = lax.broadcasted_iota(jnp.int32, (32, 128), 0) % REPLICAS
  idx_ref[...] = idx + rep * K_TAB


def _compute_indices(pitch_track):
  pitch2d = pitch_track.reshape(32, 128)
  idx2d = pl.pallas_call(
      _index_kernel,
      out_shape=jax.ShapeDtypeStruct((32, 128), jnp.int32),
  )(pitch2d)
  return idx2d.reshape(-1)


NC, NS = 2, 16
NW = NC * NS
FRAMES_PER_W = N_FRAMES // NW
CHUNK = 2
NCHUNK = FRAMES_PER_W // CHUNK
NBUF = 4


S_GROUPS = 1
G_FRAMES = N_FRAMES // S_GROUPS
G_PER_W = G_FRAMES // NW
G_NCHUNK = G_PER_W // CHUNK


def _make_gather():
  mesh = plsc.VectorSubcoreMesh(core_axis_name="c", subcore_axis_name="s")

  @functools.partial(
      pl.kernel,
      out_type=jax.ShapeDtypeStruct((G_FRAMES, 72, 128), jnp.float32),
      name="sc_table_gather",
      mesh=mesh,
      compiler_params=pltpu.CompilerParams(use_tc_tiling_on_sc=True),
      scratch_types=(
          [pltpu.VMEM((G_NCHUNK, CHUNK), jnp.int32)]
          + [pltpu.VMEM((CHUNK, 72, 128), jnp.float32)] * NBUF
          + [pltpu.SemaphoreType.DMA] * (2 * NBUF)
      ),
  )
  def gather_k(table_hbm, idx_hbm, out_hbm, idx_v, *rest):
    bufs, gsems, wsems = rest[:NBUF], rest[NBUF:2 * NBUF], rest[2 * NBUF:]
    wid = lax.axis_index("s") * NC + lax.axis_index("c")
    base = wid * G_PER_W
    pltpu.sync_copy(idx_hbm.at[pl.ds(wid * G_NCHUNK, G_NCHUNK)], idx_v)

    def g_copy(g, b):
      return pltpu.make_async_copy(table_hbm.at[idx_v.at[g]], bufs[b], gsems[b])

    def w_copy(g, b):
      return pltpu.make_async_copy(
          bufs[b], out_hbm.at[pl.ds(base + g * CHUNK, CHUNK)], wsems[b])

    for b in range(NBUF):
      g_copy(b, b).start()

    @pl.loop(0, G_NCHUNK - NBUF, step=NBUF)
    def _main(g0):
      for b in range(NBUF):
        g = g0 + b
        g_copy(g, b).wait()
        w_copy(g, b).start()
      for b in range(NBUF):
        g = g0 + b
        w_copy(g, b).wait()
        g_copy(g + NBUF, b).start()

    for b in range(NBUF):
      g = G_NCHUNK - NBUF + b
      g_copy(g, b).wait()
      w_copy(g, b).start()
    for b in range(NBUF):
      g = G_NCHUNK - NBUF + b
      w_copy(g, b).wait()

  return gather_k


_FB = 64


def _compact_first(part_ref, out_ref):
  out_ref[...] = part_ref[:, :18, :]


def _compact_next(prev_ref, part_ref, out_ref):
  out_ref[...] = part_ref[:, :18, :]


def _compact(part, prev, group):
  out_shape = jax.ShapeDtypeStruct((N_FRAMES, 18, 512), jnp.float32)
  row_blk = group * (G_FRAMES // _FB)
  out_spec = pl.BlockSpec((_FB, 18, 512), lambda i: (i + row_blk, 0, 0))
  part_spec = pl.BlockSpec((_FB, 24, 512), lambda i: (i, 0, 0))
  if prev is None:
    return pl.pallas_call(
        _compact_first,
        grid=(G_FRAMES // _FB,),
        in_specs=[part_spec],
        out_specs=out_spec,
        out_shape=out_shape,
    )(part)
  return pl.pallas_call(
      _compact_next,
      grid=(G_FRAMES // _FB,),
      in_specs=[pl.BlockSpec(memory_space=pl.ANY), part_spec],
      out_specs=out_spec,
      out_shape=out_shape,
      input_output_aliases={0: 0},
  )(prev, part)


_gather_cache = []


def kernel(pitch_track, latent_selection):
  if not _gather_cache:
    _gather_cache.append(_make_gather())
  gather_k = _gather_cache[0]
  idx = _compute_indices(pitch_track)
  table_p = latent_selection.reshape(K_TAB, 72, 128)
  table_p = jnp.tile(table_p, (REPLICAS, 1, 1))
  if S_GROUPS == 1:
    out_p = gather_k(table_p, idx.reshape(-1, CHUNK))
    return out_p.reshape(N_FRAMES, 18, 512)
  out = None
  for s in range(S_GROUPS):
    idx_g = idx[s * G_FRAMES:(s + 1) * G_FRAMES].reshape(-1, CHUNK)
    part = gather_k(table_p, idx_g)
    out = _compact(part, out, s)
  return out


# speedup vs baseline: 1.0034x; 1.0034x over previous
import functools

import jax
import jax.numpy as jnp
from jax import lax
from jax.experimental import pallas as pl
from jax.experimental.pallas import tpu as pltpu
from jax.experimental.pallas import tpu_sc as plsc

N_FRAMES = 4096
K_TAB = 16
D_ROW = 18 * 512
REPLICAS = 8


import numpy as np

_I32_MIN = np.int32(-2147483648)
_RANKS = (1023, 1024, 3071, 3072)


def _key_to_float(v):
  b = jnp.where(v >= 0, v, _I32_MIN - v)
  return lax.bitcast_convert_type(b, jnp.float32)


def _index_kernel(pitch_ref, idx_ref):
  x = pitch_ref[...]
  b = lax.bitcast_convert_type(x, jnp.int32)
  k = jnp.where(b >= 0, b, _I32_MIN - b)

  stats = []
  for r in _RANKS:
    r = jnp.int32(r)
    cnt0 = jnp.sum((k < 0).astype(jnp.int32))
    ans = jnp.where(cnt0 <= r, jnp.int32(0), _I32_MIN)
    for bit in range(30, -1, -1):
      t = ans + jnp.int32(1 << bit)
      cnt = jnp.sum((k < t).astype(jnp.int32))
      ans = jnp.where(cnt <= r, t, ans)
    stats.append(_key_to_float(ans))
  s1023, s1024, s3071, s3072 = stats

  low = s1023 * jnp.float32(0.25) + s1024 * jnp.float32(0.75)
  high = s3071 * jnp.float32(0.75) + s3072 * jnp.float32(0.25)

  pt = x - low
  pt = pt / high
  pt = pt * jnp.float32(16.0)
  m = pt - jnp.float32(16.0) * jnp.floor(pt * jnp.float32(0.0625))
  idx = jnp.round(m).astype(jnp.int32) % K_TAB
  rep = lax.broadcasted_iota(jnp.int32, (32, 128), 0) % REPLICAS
  idx_ref[...] = idx + rep * K_TAB


def _compute_indices(pitch_track):
  pitch2d = pitch_track.reshape(32, 128)
  idx2d = pl.pallas_call(
      _index_kernel,
      out_shape=jax.ShapeDtypeStruct((32, 128), jnp.int32),
  )(pitch2d)
  return idx2d.reshape(-1)


NC, NS = 2, 16
NW = NC * NS
FRAMES_PER_W = N_FRAMES // NW
CHUNK = 2
NCHUNK = FRAMES_PER_W // CHUNK
NBUF = 4


S_GROUPS = 1
G_FRAMES = N_FRAMES // S_GROUPS
G_PER_W = G_FRAMES // NW
G_NCHUNK = G_PER_W // CHUNK


def _make_gather():
  mesh = plsc.VectorSubcoreMesh(core_axis_name="c", subcore_axis_name="s")

  @functools.partial(
      pl.kernel,
      out_type=jax.ShapeDtypeStruct((G_FRAMES, D_ROW), jnp.float32),
      name="sc_table_gather",
      mesh=mesh,
      scratch_types=(
          [pltpu.VMEM((G_NCHUNK, CHUNK), jnp.int32)]
          + [pltpu.VMEM((CHUNK, D_ROW), jnp.float32)] * NBUF
          + [pltpu.SemaphoreType.DMA] * (2 * NBUF)
      ),
  )
  def gather_k(table_hbm, idx_hbm, out_hbm, idx_v, *rest):
    bufs, gsems, wsems = rest[:NBUF], rest[NBUF:2 * NBUF], rest[2 * NBUF:]
    wid = lax.axis_index("s") * NC + lax.axis_index("c")
    base = wid * G_PER_W
    pltpu.sync_copy(idx_hbm.at[pl.ds(wid * G_NCHUNK, G_NCHUNK)], idx_v)

    def g_copy(g, b):
      return pltpu.make_async_copy(table_hbm.at[idx_v.at[g]], bufs[b], gsems[b])

    def w_copy(g, b):
      return pltpu.make_async_copy(
          bufs[b], out_hbm.at[pl.ds(base + g * CHUNK, CHUNK)], wsems[b])

    for b in range(NBUF):
      g_copy(b, b).start()

    @pl.loop(0, G_NCHUNK - NBUF, step=NBUF)
    def _main(g0):
      for b in range(NBUF):
        g = g0 + b
        g_copy(g, b).wait()
        w_copy(g, b).start()
      for b in range(NBUF):
        g = g0 + b
        w_copy(g, b).wait()
        g_copy(g + NBUF, b).start()

    for b in range(NBUF):
      g = G_NCHUNK - NBUF + b
      g_copy(g, b).wait()
      w_copy(g, b).start()
    for b in range(NBUF):
      g = G_NCHUNK - NBUF + b
      w_copy(g, b).wait()

  return gather_k


_FB = 64


def _compact_first(part_ref, out_ref):
  out_ref[...] = part_ref[:, :18, :]


def _compact_next(prev_ref, part_ref, out_ref):
  out_ref[...] = part_ref[:, :18, :]


def _compact(part, prev, group):
  out_shape = jax.ShapeDtypeStruct((N_FRAMES, 18, 512), jnp.float32)
  row_blk = group * (G_FRAMES // _FB)
  out_spec = pl.BlockSpec((_FB, 18, 512), lambda i: (i + row_blk, 0, 0))
  part_spec = pl.BlockSpec((_FB, 24, 512), lambda i: (i, 0, 0))
  if prev is None:
    return pl.pallas_call(
        _compact_first,
        grid=(G_FRAMES // _FB,),
        in_specs=[part_spec],
        out_specs=out_spec,
        out_shape=out_shape,
    )(part)
  return pl.pallas_call(
      _compact_next,
      grid=(G_FRAMES // _FB,),
      in_specs=[pl.BlockSpec(memory_space=pl.ANY), part_spec],
      out_specs=out_spec,
      out_shape=out_shape,
      input_output_aliases={0: 0},
  )(prev, part)


_gather_cache = []


def kernel(pitch_track, latent_selection):
  if not _gather_cache:
    _gather_cache.append(_make_gather())
  gather_k = _gather_cache[0]
  idx = _compute_indices(pitch_track)
  table_p = latent_selection.reshape(K_TAB, D_ROW)
  table_p = jnp.tile(table_p, (REPLICAS, 1))
  if S_GROUPS == 1:
    out_p = gather_k(table_p, idx.reshape(-1, CHUNK))
    return out_p.reshape(N_FRAMES, 18, 512)
  out = None
  for s in range(S_GROUPS):
    idx_g = idx[s * G_FRAMES:(s + 1) * G_FRAMES].reshape(-1, CHUNK)
    part = gather_k(table_p, idx_g)
    out = _compact(part, out, s)
  return out


# speedup vs baseline: 1.3215x; 1.3170x over previous
import functools

import jax
import jax.numpy as jnp
from jax import lax
from jax.experimental import pallas as pl
from jax.experimental.pallas import tpu as pltpu
from jax.experimental.pallas import tpu_sc as plsc

N_FRAMES = 4096
K_TAB = 16
D_ROW = 18 * 512
REPLICAS = 16


import numpy as np

_I32_MIN = np.int32(-2147483648)
_RANKS = (1023, 1024, 3071, 3072)


def _key_to_float(v):
  b = jnp.where(v >= 0, v, _I32_MIN - v)
  return lax.bitcast_convert_type(b, jnp.float32)


def _index_kernel(pitch_ref, idx_ref):
  x = pitch_ref[...]
  b = lax.bitcast_convert_type(x, jnp.int32)
  k = jnp.where(b >= 0, b, _I32_MIN - b)

  stats = []
  for r in _RANKS:
    r = jnp.int32(r)
    cnt0 = jnp.sum((k < 0).astype(jnp.int32))
    ans = jnp.where(cnt0 <= r, jnp.int32(0), _I32_MIN)
    for bit in range(30, -1, -1):
      t = ans + jnp.int32(1 << bit)
      cnt = jnp.sum((k < t).astype(jnp.int32))
      ans = jnp.where(cnt <= r, t, ans)
    stats.append(_key_to_float(ans))
  s1023, s1024, s3071, s3072 = stats

  low = s1023 * jnp.float32(0.25) + s1024 * jnp.float32(0.75)
  high = s3071 * jnp.float32(0.75) + s3072 * jnp.float32(0.25)

  pt = x - low
  pt = pt / high
  pt = pt * jnp.float32(16.0)
  m = pt - jnp.float32(16.0) * jnp.floor(pt * jnp.float32(0.0625))
  idx = jnp.round(m).astype(jnp.int32) % K_TAB
  rep = lax.broadcasted_iota(jnp.int32, (32, 128), 0) % REPLICAS
  idx_ref[...] = idx + rep * K_TAB


def _compute_indices(pitch_track):
  pitch2d = pitch_track.reshape(32, 128)
  idx2d = pl.pallas_call(
      _index_kernel,
      out_shape=jax.ShapeDtypeStruct((32, 128), jnp.int32),
  )(pitch2d)
  return idx2d.reshape(-1)


NC, NS = 2, 16
NW = NC * NS
FRAMES_PER_W = N_FRAMES // NW
CHUNK = 2
NCHUNK = FRAMES_PER_W // CHUNK
NBUF = 4


S_GROUPS = 1
G_FRAMES = N_FRAMES // S_GROUPS
G_PER_W = G_FRAMES // NW
G_NCHUNK = G_PER_W // CHUNK


def _make_gather():
  mesh = plsc.VectorSubcoreMesh(core_axis_name="c", subcore_axis_name="s")

  @functools.partial(
      pl.kernel,
      out_type=jax.ShapeDtypeStruct((G_FRAMES, 24, 512), jnp.float32),
      name="sc_table_gather",
      mesh=mesh,
      compiler_params=pltpu.CompilerParams(use_tc_tiling_on_sc=True),
      scratch_types=(
          [pltpu.VMEM((G_NCHUNK, CHUNK), jnp.int32)]
          + [pltpu.VMEM((CHUNK, 24, 512), jnp.float32)] * NBUF
          + [pltpu.SemaphoreType.DMA] * (2 * NBUF)
      ),
  )
  def gather_k(table_hbm, idx_hbm, out_hbm, idx_v, *rest):
    bufs, gsems, wsems = rest[:NBUF], rest[NBUF:2 * NBUF], rest[2 * NBUF:]
    wid = lax.axis_index("s") * NC + lax.axis_index("c")
    base = wid * G_PER_W
    pltpu.sync_copy(idx_hbm.at[pl.ds(wid * G_NCHUNK, G_NCHUNK)], idx_v)

    def g_copy(g, b):
      return pltpu.make_async_copy(table_hbm.at[idx_v.at[g]], bufs[b], gsems[b])

    def w_copy(g, b):
      return pltpu.make_async_copy(
          bufs[b], out_hbm.at[pl.ds(base + g * CHUNK, CHUNK)], wsems[b])

    for b in range(NBUF):
      g_copy(b, b).start()

    @pl.loop(0, G_NCHUNK - NBUF, step=NBUF)
    def _main(g0):
      for b in range(NBUF):
        g = g0 + b
        g_copy(g, b).wait()
        w_copy(g, b).start()
      for b in range(NBUF):
        g = g0 + b
        w_copy(g, b).wait()
        g_copy(g + NBUF, b).start()

    for b in range(NBUF):
      g = G_NCHUNK - NBUF + b
      g_copy(g, b).wait()
      w_copy(g, b).start()
    for b in range(NBUF):
      g = G_NCHUNK - NBUF + b
      w_copy(g, b).wait()

  return gather_k


_FB = 64


def _compact_first(part_ref, out_ref):
  out_ref[...] = part_ref[:, :18, :]


def _compact_next(prev_ref, part_ref, out_ref):
  out_ref[...] = part_ref[:, :18, :]


def _compact(part, prev, group):
  out_shape = jax.ShapeDtypeStruct((N_FRAMES, 18, 512), jnp.float32)
  row_blk = group * (G_FRAMES // _FB)
  out_spec = pl.BlockSpec((_FB, 18, 512), lambda i: (i + row_blk, 0, 0))
  part_spec = pl.BlockSpec((_FB, 24, 512), lambda i: (i, 0, 0))
  if prev is None:
    return pl.pallas_call(
        _compact_first,
        grid=(G_FRAMES // _FB,),
        in_specs=[part_spec],
        out_specs=out_spec,
        out_shape=out_shape,
    )(part)
  return pl.pallas_call(
      _compact_next,
      grid=(G_FRAMES // _FB,),
      in_specs=[pl.BlockSpec(memory_space=pl.ANY), part_spec],
      out_specs=out_spec,
      out_shape=out_shape,
      input_output_aliases={0: 0},
  )(prev, part)


_gather_cache = []


def kernel(pitch_track, latent_selection):
  if not _gather_cache:
    _gather_cache.append(_make_gather())
  gather_k = _gather_cache[0]
  idx = _compute_indices(pitch_track)
  table_p = jnp.pad(latent_selection, ((0, 0), (0, 6), (0, 0)))
  table_p = jnp.tile(table_p, (REPLICAS, 1, 1))
  if S_GROUPS == 1:
    out_p = gather_k(table_p, idx.reshape(-1, CHUNK))
    return out_p[:, :18, :]
  out = None
  for s in range(S_GROUPS):
    idx_g = idx[s * G_FRAMES:(s + 1) * G_FRAMES].reshape(-1, CHUNK)
    part = gather_k(table_p, idx_g)
    out = _compact(part, out, s)
  return out


# speedup vs baseline: 1.3471x; 1.0193x over previous
import functools

import jax
import jax.numpy as jnp
from jax import lax
from jax.experimental import pallas as pl
from jax.experimental.pallas import tpu as pltpu
from jax.experimental.pallas import tpu_sc as plsc

N_FRAMES = 4096
K_TAB = 16
D_ROW = 18 * 512
REPLICAS = 8


import numpy as np

_I32_MIN = np.int32(-2147483648)
_RANKS = (1023, 1024, 3071, 3072)


def _key_to_float(v):
  b = jnp.where(v >= 0, v, _I32_MIN - v)
  return lax.bitcast_convert_type(b, jnp.float32)


def _index_kernel(pitch_ref, idx_ref):
  x = pitch_ref[...]
  b = lax.bitcast_convert_type(x, jnp.int32)
  k = jnp.where(b >= 0, b, _I32_MIN - b)

  stats = []
  for r in _RANKS:
    r = jnp.int32(r)
    cnt0 = jnp.sum((k < 0).astype(jnp.int32))
    ans = jnp.where(cnt0 <= r, jnp.int32(0), _I32_MIN)
    for bit in range(30, -1, -1):
      t = ans + jnp.int32(1 << bit)
      cnt = jnp.sum((k < t).astype(jnp.int32))
      ans = jnp.where(cnt <= r, t, ans)
    stats.append(_key_to_float(ans))
  s1023, s1024, s3071, s3072 = stats

  low = s1023 * jnp.float32(0.25) + s1024 * jnp.float32(0.75)
  high = s3071 * jnp.float32(0.75) + s3072 * jnp.float32(0.25)

  pt = x - low
  pt = pt / high
  pt = pt * jnp.float32(16.0)
  m = pt - jnp.float32(16.0) * jnp.floor(pt * jnp.float32(0.0625))
  idx = jnp.round(m).astype(jnp.int32) % K_TAB
  rep = lax.broadcasted_iota(jnp.int32, (32, 128), 0) % REPLICAS
  idx_ref[...] = idx + rep * K_TAB


def _compute_indices(pitch_track):
  pitch2d = pitch_track.reshape(32, 128)
  idx2d = pl.pallas_call(
      _index_kernel,
      out_shape=jax.ShapeDtypeStruct((32, 128), jnp.int32),
  )(pitch2d)
  return idx2d.reshape(-1)


NC, NS = 2, 16
NW = NC * NS
FRAMES_PER_W = N_FRAMES // NW
CHUNK = 4
NCHUNK = FRAMES_PER_W // CHUNK
NBUF = 2


S_GROUPS = 1
G_FRAMES = N_FRAMES // S_GROUPS
G_PER_W = G_FRAMES // NW
G_NCHUNK = G_PER_W // CHUNK


def _make_gather():
  mesh = plsc.VectorSubcoreMesh(core_axis_name="c", subcore_axis_name="s")

  @functools.partial(
      pl.kernel,
      out_type=jax.ShapeDtypeStruct((G_FRAMES, 24, 512), jnp.float32),
      name="sc_table_gather",
      mesh=mesh,
      compiler_params=pltpu.CompilerParams(use_tc_tiling_on_sc=True),
      scratch_types=(
          [pltpu.VMEM((G_NCHUNK, CHUNK), jnp.int32)]
          + [pltpu.VMEM((CHUNK, 24, 512), jnp.float32)] * NBUF
          + [pltpu.SemaphoreType.DMA] * NBUF
      ),
  )
  def gather_k(table_hbm, idx_hbm, out_hbm, idx_v, *rest):
    bufs, gsems = rest[:NBUF], rest[NBUF:]
    wid = lax.axis_index("s") * NC + lax.axis_index("c")
    base = wid * G_PER_W
    pltpu.sync_copy(idx_hbm.at[pl.ds(wid * G_NCHUNK, G_NCHUNK)], idx_v)

    def g_copy(g, b):
      return pltpu.make_async_copy(table_hbm.at[idx_v.at[g]], bufs[b], gsems[b])

    def put_out(g, b):
      pltpu.sync_copy(bufs[b], out_hbm.at[pl.ds(base + g * CHUNK, CHUNK)])

    for b in range(NBUF):
      g_copy(b, b).start()

    @pl.loop(0, G_NCHUNK - NBUF, step=NBUF)
    def _main(g0):
      for b in range(NBUF):
        g = g0 + b
        g_copy(g, b).wait()
        put_out(g, b)
        g_copy(g + NBUF, b).start()

    for b in range(NBUF):
      g = G_NCHUNK - NBUF + b
      g_copy(g, b).wait()
      put_out(g, b)

  return gather_k


_FB = 64


def _compact_first(part_ref, out_ref):
  out_ref[...] = part_ref[:, :18, :]


def _compact_next(prev_ref, part_ref, out_ref):
  out_ref[...] = part_ref[:, :18, :]


def _compact(part, prev, group):
  out_shape = jax.ShapeDtypeStruct((N_FRAMES, 18, 512), jnp.float32)
  row_blk = group * (G_FRAMES // _FB)
  out_spec = pl.BlockSpec((_FB, 18, 512), lambda i: (i + row_blk, 0, 0))
  part_spec = pl.BlockSpec((_FB, 24, 512), lambda i: (i, 0, 0))
  if prev is None:
    return pl.pallas_call(
        _compact_first,
        grid=(G_FRAMES // _FB,),
        in_specs=[part_spec],
        out_specs=out_spec,
        out_shape=out_shape,
    )(part)
  return pl.pallas_call(
      _compact_next,
      grid=(G_FRAMES // _FB,),
      in_specs=[pl.BlockSpec(memory_space=pl.ANY), part_spec],
      out_specs=out_spec,
      out_shape=out_shape,
      input_output_aliases={0: 0},
  )(prev, part)


_gather_cache = []


def kernel(pitch_track, latent_selection):
  if not _gather_cache:
    _gather_cache.append(_make_gather())
  gather_k = _gather_cache[0]
  idx = _compute_indices(pitch_track)
  table_p = jnp.pad(latent_selection, ((0, 0), (0, 6), (0, 0)))
  table_p = jnp.tile(table_p, (REPLICAS, 1, 1))
  if S_GROUPS == 1:
    out_p = gather_k(table_p, idx.reshape(-1, CHUNK))
    return out_p[:, :18, :]
  out = None
  for s in range(S_GROUPS):
    idx_g = idx[s * G_FRAMES:(s + 1) * G_FRAMES].reshape(-1, CHUNK)
    part = gather_k(table_p, idx_g)
    out = _compact(part, out, s)
  return out
